# Initial kernel scaffold; baseline (speedup 1.0000x reference)
import jax
import jax.numpy as jnp
from jax.experimental import pallas as pl


def kernel(model_activations, W_enc, b_enc, W_dec, b_dec, W_enc1, b_enc1, W_dec1, b_dec1, W_enc2, b_enc2, W_dec2, b_dec2):
    raise NotImplementedError("write your pallas kernel here")


# speedup vs baseline: 2.4244x; 2.4244x over previous
import functools

import jax
import jax.numpy as jnp
from jax import lax
from jax.experimental import pallas as pl
from jax.experimental.pallas import tpu as pltpu
from jax.experimental.pallas import tpu_sc as plsc

B = 2048
D = 2048
SAE = 8192
K = 3


BB = 256
SB = 1024
NRB = B // BB
NSB = SAE // SB
NEG_INF = float("-inf")


def _top3_of_tile(pre, j):
    iota = lax.broadcasted_iota(jnp.int32, (BB, SB), 1)
    t = pre
    vs, is_ = [], []
    for _ in range(3):
        m = jnp.max(t, axis=1, keepdims=True)
        pos = jnp.min(jnp.where(t == m, iota, SB), axis=1, keepdims=True)
        vs.append(m)
        is_.append(pos + j * SB)
        t = jnp.where(iota == pos, NEG_INF, t)
    return jnp.concatenate(vs, axis=1), jnp.concatenate(is_, axis=1)


def _merge6(v6, i6):
    iota = lax.broadcasted_iota(jnp.int32, (BB, 6), 1)
    vs, is_ = [], []
    for _ in range(3):
        m = jnp.max(v6, axis=1, keepdims=True)
        pos = jnp.min(jnp.where(v6 == m, iota, 6), axis=1, keepdims=True)
        vs.append(m)
        is_.append(jnp.sum(jnp.where(iota == pos, i6, 0), axis=1, keepdims=True))
        v6 = jnp.where(iota == pos, NEG_INF, v6)
    return jnp.concatenate(vs, axis=1), jnp.concatenate(is_, axis=1)


def _topk_body(x_ref, w_ref, b_ref, vals_ref, idx_ref, tv, ti):
    j = pl.program_id(0)
    i = pl.program_id(1)
    pre = lax.dot_general(x_ref[...], w_ref[...], (((1,), (1,)), ((), ())),
                          preferred_element_type=jnp.float32) + b_ref[0]
    nv, ni = _top3_of_tile(pre, j)
    rows = pl.ds(i * BB, BB)

    @pl.when(j == 0)
    def _():
        tv[rows, :] = nv
        ti[rows, :] = ni

    @pl.when(j > 0)
    def _():
        mv, mi = _merge6(jnp.concatenate([tv[rows, :], nv], axis=1),
                         jnp.concatenate([ti[rows, :], ni], axis=1))
        tv[rows, :] = mv
        ti[rows, :] = mi

    @pl.when(j == NSB - 1)
    def _():
        zf = jnp.zeros((BB, 13), jnp.float32)
        zi = jnp.zeros((BB, 13), jnp.int32)
        vals_ref[...] = jnp.concatenate([tv[rows, :], zf], axis=1)
        idx_ref[...] = jnp.concatenate([ti[rows, :], zi], axis=1)


def _topk_call(x, w_enc, b_enc2d):
    return pl.pallas_call(
        _topk_body,
        grid=(NSB, NRB),
        in_specs=[
            pl.BlockSpec((BB, D), lambda j, i: (i, 0)),
            pl.BlockSpec((SB, D), lambda j, i: (j, 0)),
            pl.BlockSpec((1, 1, SB), lambda j, i: (j, 0, 0)),
        ],
        out_specs=[
            pl.BlockSpec((BB, 16), lambda j, i: (i, 0)),
            pl.BlockSpec((BB, 16), lambda j, i: (i, 0)),
        ],
        out_shape=[
            jax.ShapeDtypeStruct((B, 16), jnp.float32),
            jax.ShapeDtypeStruct((B, 16), jnp.int32),
        ],
        scratch_shapes=[
            pltpu.VMEM((B, 3), jnp.float32),
            pltpu.VMEM((B, 3), jnp.int32),
        ],
    )(x, w_enc, b_enc2d)


NC = 2
NS = 16
NW = NC * NS
RPW = B // NW
NCH = D // 16


def _sc_body(x_hbm, idx_hbm, vals_hbm, we1_hbm, we2_hbm,
             wd_hbm, wd1_hbm, wd2_hbm, be1_hbm, be2_hbm, bsum_hbm, zero_hbm,
             recon_hbm, bm_hbm,
             xrow_v, idx_v, vals_v, w1r, w2r, d0r, d1r, d2r,
             outrow_v, be1_v, be2_v, bsum_v, bm_v, red_v, sem):
    wid = lax.axis_index("s") * NC + lax.axis_index("c")
    pltpu.sync_copy(be1_hbm, be1_v)
    pltpu.sync_copy(be2_hbm, be2_v)
    pltpu.sync_copy(bsum_hbm, bsum_v)
    pltpu.sync_copy(zero_hbm, bm_v)
    lane = lax.iota(jnp.int32, 16)
    zero16 = jnp.zeros((16,), jnp.float32)
    red_v[pl.ds(16, 16)] = zero16

    def hsum(v):
        for sh in (8, 4, 2, 1):
            red_v[pl.ds(0, 16)] = v
            v = red_v[pl.ds(0, 16)] + red_v[pl.ds(sh, 16)]
        return v[0]

    def row_body(r, carry):
        b = wid * RPW + r
        pltpu.sync_copy(x_hbm.at[pl.ds(b * D, D)], xrow_v)
        pltpu.sync_copy(idx_hbm.at[pl.ds(b * 16, 16)], idx_v)
        pltpu.sync_copy(vals_hbm.at[pl.ds(b * 16, 16)], vals_v)
        idx3 = idx_v.at[pl.ds(0, 3)]
        cps = [pltpu.async_copy(we1_hbm.at[idx3], w1r, sem),
               pltpu.async_copy(we2_hbm.at[idx3], w2r, sem),
               pltpu.async_copy(wd_hbm.at[idx3], d0r, sem),
               pltpu.async_copy(wd1_hbm.at[idx3], d1r, sem),
               pltpu.async_copy(wd2_hbm.at[idx3], d2r, sem)]
        for cp in cps:
            cp.wait()

        def dot_body(c, accs):
            sl = pl.ds(c * 16, 16)
            xc = xrow_v[sl]
            return (accs[0] + xc * w1r[0, sl], accs[1] + xc * w1r[1, sl],
                    accs[2] + xc * w1r[2, sl], accs[3] + xc * w2r[0, sl],
                    accs[4] + xc * w2r[1, sl], accs[5] + xc * w2r[2, sl])

        accs = lax.fori_loop(0, NCH, dot_body, (zero16,) * 6)

        idx_vec = idx_v[...]
        vals_vec = vals_v[...]
        cv = []
        for k in range(3):
            ik = idx_vec[k]
            w1k = hsum(accs[k]) + be1_v[pl.ds(ik, 16)][0]
            w2k = hsum(accs[3 + k]) + be2_v[pl.ds(ik, 16)][0]
            valk = vals_vec[k]
            live = valk != 0.0
            win = w1k > w2k
            c1k = jnp.where(live & win, w1k, 0.0)
            c2k = jnp.where(live & (~win), w2k, 0.0)
            cv.append((valk, c1k, c2k))

        def rec_body(c, carry):
            sl = pl.ds(c * 16, 16)
            o = bsum_v[sl]
            o = o + cv[0][0] * d0r[0, sl] + cv[1][0] * d0r[1, sl] + cv[2][0] * d0r[2, sl]
            o = o + cv[0][1] * d1r[0, sl] + cv[1][1] * d1r[1, sl] + cv[2][1] * d1r[2, sl]
            o = o + cv[0][2] * d2r[0, sl] + cv[1][2] * d2r[1, sl] + cv[2][2] * d2r[2, sl]
            outrow_v[sl] = o
            return 0

        lax.fori_loop(0, NCH, rec_body, 0)
        pltpu.sync_copy(outrow_v, recon_hbm.at[pl.ds(b * D, D)])

        for k in range(3):
            ik = idx_vec[k]
            for ch, flag in ((0, jnp.float32(1.0)),
                             (1, jnp.where(cv[k][1] != 0.0, 1.0, 0.0)),
                             (2, jnp.where(cv[k][2] != 0.0, 1.0, 0.0))):
                sl = pl.ds(ch * (SAE + 16) + ik, 16)
                cur = bm_v[sl]
                bm_v[sl] = jnp.where(lane == 0,
                                     jnp.maximum(cur, flag), cur)
        return 0

    lax.fori_loop(0, RPW, row_body, 0)
    for ch in range(3):
        pltpu.sync_copy(bm_v.at[pl.ds(ch * (SAE + 16), SAE)],
                        bm_hbm.at[pl.ds((wid * 3 + ch) * SAE, SAE)])


@functools.cache
def _get_sc_call():
    return functools.partial(
        pl.kernel,
        mesh=plsc.VectorSubcoreMesh(core_axis_name="c", subcore_axis_name="s"),
        compiler_params=pltpu.CompilerParams(use_tc_tiling_on_sc=False),
        out_type=[jax.ShapeDtypeStruct((B * D,), jnp.float32),
                  jax.ShapeDtypeStruct((NW * 3 * SAE,), jnp.float32)],
        scratch_types=[
            pltpu.VMEM((D,), jnp.float32),
            pltpu.VMEM((16,), jnp.int32),
            pltpu.VMEM((16,), jnp.float32),
            pltpu.VMEM((3, D), jnp.float32),
            pltpu.VMEM((3, D), jnp.float32),
            pltpu.VMEM((3, D), jnp.float32),
            pltpu.VMEM((3, D), jnp.float32),
            pltpu.VMEM((3, D), jnp.float32),
            pltpu.VMEM((D,), jnp.float32),
            pltpu.VMEM((SAE + 16,), jnp.float32),
            pltpu.VMEM((SAE + 16,), jnp.float32),
            pltpu.VMEM((D,), jnp.float32),
            pltpu.VMEM((3 * (SAE + 16),), jnp.float32),
            pltpu.VMEM((32,), jnp.float32),
            pltpu.SemaphoreType.DMA,
        ],
    )(_sc_body)


def _count_body(bm_ref, out_ref):
    mx = jnp.max(bm_ref[...], axis=0)
    cnt = jnp.sum(jnp.where(mx > 0.0, 1, 0).astype(jnp.int32),
                  axis=1, keepdims=True)
    out_ref[...] = jnp.broadcast_to(cnt, (3, 128))


def _count_call(bm):
    return pl.pallas_call(
        _count_body,
        out_shape=jax.ShapeDtypeStruct((3, 128), jnp.int32),
    )(bm)


def kernel(model_activations, W_enc, b_enc, W_dec, b_dec,
           W_enc1, b_enc1, W_dec1, b_dec1,
           W_enc2, b_enc2, W_dec2, b_dec2):
    x = model_activations
    vals16, idx16 = _topk_call(x, W_enc, b_enc.reshape(NSB, 1, SB))
    wdT = W_dec.T
    wd1T = W_dec1.T
    wd2T = W_dec2.T
    bsum = b_dec + b_dec1 + b_dec2
    zero = jnp.zeros((3 * (SAE + 16),), jnp.float32)
    pad16 = jnp.zeros((16,), jnp.float32)
    be1p = jnp.concatenate([b_enc1, pad16])
    be2p = jnp.concatenate([b_enc2, pad16])
    recon, bm = _get_sc_call()(x.reshape(-1), idx16.reshape(-1),
                               vals16.reshape(-1), W_enc1, W_enc2,
                               wdT, wd1T, wd2T, be1p, be2p, bsum, zero)
    cnt = _count_call(bm.reshape(NW, 3, SAE))
    return (recon.reshape(B, D), (cnt[0, 0], cnt[1, 0], cnt[2, 0]))
